# Initial kernel scaffold; baseline (speedup 1.0000x reference)
import jax
import jax.numpy as jnp
from jax.experimental import pallas as pl


def kernel(node_feat, edge_feat, edge_index):
    raise NotImplementedError("write your pallas kernel here")


# speedup vs baseline: 1.9083x; 1.9083x over previous
import functools

import jax
import jax.numpy as jnp
from jax import lax
from jax.experimental import pallas as pl
from jax.experimental.pallas import tpu as pltpu
from jax.experimental.pallas import tpu_sc as plsc

_TILES = 16
_B = 80


def kernel(node_feat, edge_feat, edge_index):
    n, d = node_feat.shape
    e = edge_feat.shape[0]
    h = d // 2

    node2 = node_feat.reshape(2 * n, h)
    edge2 = edge_feat.reshape(2 * e, h)
    src = edge_index[0]
    dst = edge_index[1]

    epw = e // _TILES
    nchunks = epw // _B
    n_pad = ((n + _TILES * 128 - 1) // (_TILES * 128)) * (_TILES * 128)
    zrows = 128
    rows_pt = n_pad // _TILES
    nz = rows_pt // zrows

    mesh = plsc.VectorSubcoreMesh(core_axis_name="c", subcore_axis_name="s")

    @functools.partial(
        pl.kernel,
        out_type=jax.ShapeDtypeStruct((2, n_pad, h), jnp.float32),
        mesh=mesh,
        scratch_types=[
            pltpu.VMEM((_B,), jnp.int32),
            pltpu.VMEM((1, _B), jnp.int32),
            pltpu.VMEM((1, _B), jnp.int32),
            pltpu.VMEM((1, _B), jnp.int32),
            pltpu.VMEM((_B, h), jnp.float32),
            pltpu.VMEM((_B, h), jnp.float32),
            pltpu.VMEM((zrows, h), jnp.float32),
            pltpu.VMEM_SHARED((n_pad, h), jnp.float32),
        ],
    )
    def k(node_hbm, edge_hbm, src_hbm, dst_hbm, out_hbm,
          srcv, nidx, eidx, didx, nbuf, ebuf, zbuf, acc):
        c = lax.axis_index("c")
        s = lax.axis_index("s")

        zeros = jnp.zeros((16,), jnp.float32)

        @pl.loop(0, zrows)
        def _(r):
            @pl.loop(0, h, step=16)
            def _(j):
                zbuf[r, pl.ds(j, 16)] = zeros

        @pl.loop(0, nz)
        def _(m):
            pltpu.sync_copy(zbuf, acc.at[pl.ds(s * rows_pt + m * zrows, zrows)])

        plsc.subcore_barrier()

        base0 = s * epw

        @pl.loop(0, nchunks)
        def _(kk):
            base = base0 + kk * _B
            pltpu.sync_copy(src_hbm.at[pl.ds(base, _B)], srcv)
            pltpu.sync_copy(dst_hbm.at[pl.ds(base, _B)], didx.at[0])

            @pl.loop(0, _B, step=16)
            def _(j):
                sv = srcv[pl.ds(j, 16)]
                nidx[0, pl.ds(j, 16)] = sv * 2 + c
                ii = lax.iota(jnp.int32, 16) + (base + j)
                eidx[0, pl.ds(j, 16)] = ii * 2 + c

            pltpu.sync_copy(node_hbm.at[nidx.at[0]], nbuf)
            pltpu.sync_copy(edge_hbm.at[eidx.at[0]], ebuf)
            pltpu.sync_copy(nbuf, acc.at[didx.at[0]], add=True)
            pltpu.sync_copy(ebuf, acc.at[didx.at[0]], add=True)

        plsc.subcore_barrier()
        pltpu.sync_copy(acc.at[pl.ds(s * rows_pt, rows_pt)],
                        out_hbm.at[c].at[pl.ds(s * rows_pt, rows_pt)])

    out3 = k(node2, edge2, src, dst)
    return out3[:, :n, :].transpose(1, 0, 2).reshape(n, d)


# speedup vs baseline: 3.1434x; 1.6472x over previous
import functools

import jax
import jax.numpy as jnp
from jax import lax
from jax.experimental import pallas as pl
from jax.experimental.pallas import tpu as pltpu
from jax.experimental.pallas import tpu_sc as plsc

_TILES = 16
_B = 80


def kernel(node_feat, edge_feat, edge_index):
    n, d = node_feat.shape
    e = edge_feat.shape[0]
    h = d // 2

    node2 = node_feat.reshape(2 * n, h)
    edge2 = edge_feat.reshape(2 * e, h)
    src = edge_index[0]
    dst = edge_index[1]

    epw = e // _TILES
    nchunks = epw // _B
    njobs = 2 * nchunks
    n_pad = ((n + _TILES * 8 - 1) // (_TILES * 8)) * (_TILES * 8)
    rows_pt = n_pad // _TILES
    zeros = jnp.zeros((n_pad, h), jnp.float32)

    mesh = plsc.VectorSubcoreMesh(core_axis_name="c", subcore_axis_name="s")

    @functools.partial(
        pl.kernel,
        out_type=jax.ShapeDtypeStruct((2, n_pad, h), jnp.float32),
        mesh=mesh,
        scratch_types=[
            pltpu.VMEM((2, _B), jnp.int32),
            pltpu.VMEM((2, _B), jnp.int32),
            pltpu.VMEM((4, _B), jnp.int32),
            pltpu.VMEM((4, _B), jnp.int32),
            pltpu.VMEM((4, _B, h), jnp.float32),
            pltpu.VMEM_SHARED((n_pad, h), jnp.float32),
            pltpu.SemaphoreType.DMA,
        ]
        + [pltpu.SemaphoreType.DMA] * 4
        + [pltpu.SemaphoreType.DMA] * 4
        + [pltpu.SemaphoreType.DMA] * 2,
    )
    def k(node_hbm, edge_hbm, src_hbm, dst_hbm, zeros_hbm, out_hbm,
          sstage, dstage, gidx, didx, buf, acc, zsem,
          g0, g1, g2, g3, s0, s1, s2, s3, l0, l1):
        gsem = [g0, g1, g2, g3]
        ssem = [s0, s1, s2, s3]
        lsem = [l0, l1]
        tabs = [node_hbm, edge_hbm, node_hbm, edge_hbm]

        c = lax.axis_index("c")
        s = lax.axis_index("s")
        base0 = s * epw
        arow = s * rows_pt

        def issue_ld(k_tr, st):
            pltpu.async_copy(src_hbm.at[pl.ds(base0 + k_tr * _B, _B)],
                             sstage.at[st], lsem[st])
            pltpu.async_copy(dst_hbm.at[pl.ds(base0 + k_tr * _B, _B)],
                             dstage.at[st], lsem[st])

        def wait_ld(st):
            pltpu.make_async_copy(src_hbm.at[pl.ds(base0, _B)],
                                  sstage.at[st], lsem[st]).wait()
            pltpu.make_async_copy(dst_hbm.at[pl.ds(base0, _B)],
                                  dstage.at[st], lsem[st]).wait()

        def gen_node(b, st):
            @pl.loop(0, _B, step=16)
            def _(j):
                didx[b, pl.ds(j, 16)] = dstage[st, pl.ds(j, 16)]
                gidx[b, pl.ds(j, 16)] = sstage[st, pl.ds(j, 16)] * 2 + c

        def gen_edge(b, st, k_tr):
            ebase = (base0 + k_tr * _B) * 2 + c

            @pl.loop(0, _B, step=16)
            def _(j):
                didx[b, pl.ds(j, 16)] = dstage[st, pl.ds(j, 16)]
                gidx[b, pl.ds(j, 16)] = lax.iota(jnp.int32, 16) * 2 + (ebase + j * 2)

        def issue_g(b):
            pltpu.async_copy(tabs[b].at[gidx.at[b]], buf.at[b], gsem[b])

        def wait_g(b):
            pltpu.make_async_copy(tabs[b].at[gidx.at[b]], buf.at[b], gsem[b]).wait()

        def issue_sc(b):
            pltpu.async_copy(buf.at[b], acc.at[didx.at[b]], ssem[b], add=True)

        def wait_sc(b):
            pltpu.make_async_copy(buf.at[b], acc.at[didx.at[b]], ssem[b]).wait()

        zcp = pltpu.async_copy(zeros_hbm.at[pl.ds(arow, rows_pt)],
                               acc.at[pl.ds(arow, rows_pt)], zsem)
        issue_ld(0, 0)
        issue_ld(1, 1)
        wait_ld(0)
        gen_node(0, 0)
        issue_g(0)
        gen_edge(1, 0, 0)
        issue_g(1)
        issue_ld(2, 0)
        wait_ld(1)
        gen_node(2, 1)
        issue_g(2)
        gen_edge(3, 1, 1)
        issue_g(3)
        issue_ld(3, 1)
        zcp.wait()
        plsc.subcore_barrier()
        wait_g(0)
        issue_sc(0)
        wait_g(1)
        issue_sc(1)
        wait_g(2)
        issue_sc(2)

        @pl.loop(4, njobs - 2, step=4)
        def _(g0_tr):
            kbase = g0_tr // 2
            for b in range(4):
                k_tr = kbase + (b // 2)
                st = b // 2
                if b % 2 == 0:
                    wait_sc(b)
                    wait_ld(st)
                    gen_node(b, st)
                    issue_g(b)
                else:
                    wait_sc(b)
                    gen_edge(b, st, k_tr)
                    issue_g(b)

                    @pl.when(k_tr + 2 < nchunks)
                    def _():
                        issue_ld(k_tr + 2, st)
                bp = (b + 3) % 4
                wait_g(bp)
                issue_sc(bp)

        wait_sc(0)
        wait_ld(0)
        gen_node(0, 0)
        issue_g(0)
        wait_g(3)
        issue_sc(3)
        wait_sc(1)
        gen_edge(1, 0, nchunks - 1)
        issue_g(1)
        wait_g(0)
        issue_sc(0)
        wait_g(1)
        issue_sc(1)
        wait_sc(2)
        wait_sc(3)
        wait_sc(0)
        wait_sc(1)

        plsc.subcore_barrier()
        pltpu.sync_copy(acc.at[pl.ds(arow, rows_pt)],
                        out_hbm.at[c].at[pl.ds(arow, rows_pt)])

    out3 = k(node2, edge2, src, dst, zeros)
    return out3[:, :n, :].transpose(1, 0, 2).reshape(n, d)


# speedup vs baseline: 3.2049x; 1.0196x over previous
import functools

import jax
import jax.numpy as jnp
from jax import lax
from jax.experimental import pallas as pl
from jax.experimental.pallas import tpu as pltpu
from jax.experimental.pallas import tpu_sc as plsc

_TILES = 16
_B = 80


def kernel(node_feat, edge_feat, edge_index):
    n, d = node_feat.shape
    e = edge_feat.shape[0]
    h = d // 2

    node2 = node_feat.reshape(2 * n, h)
    edge2 = edge_feat.reshape(2 * e, h)

    epw = e // _TILES
    nchunks = epw // _B
    njobs = 2 * nchunks
    n_pad = ((n + _TILES * 8 - 1) // (_TILES * 8)) * (_TILES * 8)
    rows_pt = n_pad // _TILES
    rows_last = n - (_TILES - 1) * rows_pt

    mesh = plsc.VectorSubcoreMesh(core_axis_name="c", subcore_axis_name="s")

    @functools.partial(
        pl.kernel,
        out_type=jax.ShapeDtypeStruct((2, n_pad, h), jnp.float32),
        mesh=mesh,
        scratch_types=[
            pltpu.VMEM((2, _B), jnp.int32),
            pltpu.VMEM((2, _B), jnp.int32),
            pltpu.VMEM((4, _B), jnp.int32),
            pltpu.VMEM((4, _B), jnp.int32),
            pltpu.VMEM((4, _B, h), jnp.float32),
            pltpu.VMEM_SHARED((n_pad, h), jnp.float32),
            pltpu.SemaphoreType.DMA,
        ]
        + [pltpu.SemaphoreType.DMA] * 4
        + [pltpu.SemaphoreType.DMA] * 4
        + [pltpu.SemaphoreType.DMA] * 2,
    )
    def k(node_hbm, edge_hbm, eidx_hbm, out_hbm,
          sstage, dstage, gidx, didx, buf, acc, zsem,
          g0, g1, g2, g3, s0, s1, s2, s3, l0, l1):
        gsem = [g0, g1, g2, g3]
        ssem = [s0, s1, s2, s3]
        lsem = [l0, l1]
        tabs = [node_hbm, edge_hbm, node_hbm, edge_hbm]

        c = lax.axis_index("c")
        s = lax.axis_index("s")
        base0 = s * epw
        arow = s * rows_pt

        def issue_ld(k_tr, st):
            pltpu.async_copy(eidx_hbm.at[pl.ds(base0 + k_tr * _B, _B)],
                             sstage.at[st], lsem[st])
            pltpu.async_copy(eidx_hbm.at[pl.ds(e + base0 + k_tr * _B, _B)],
                             dstage.at[st], lsem[st])

        def wait_ld(st):
            pltpu.make_async_copy(eidx_hbm.at[pl.ds(base0, _B)],
                                  sstage.at[st], lsem[st]).wait()
            pltpu.make_async_copy(eidx_hbm.at[pl.ds(e + base0, _B)],
                                  dstage.at[st], lsem[st]).wait()

        def gen_node(b, st):
            @pl.loop(0, _B, step=16)
            def _(j):
                didx[b, pl.ds(j, 16)] = dstage[st, pl.ds(j, 16)]
                gidx[b, pl.ds(j, 16)] = sstage[st, pl.ds(j, 16)] * 2 + c

        def gen_edge(b, st, k_tr):
            ebase = (base0 + k_tr * _B) * 2 + c

            @pl.loop(0, _B, step=16)
            def _(j):
                didx[b, pl.ds(j, 16)] = dstage[st, pl.ds(j, 16)]
                gidx[b, pl.ds(j, 16)] = lax.iota(jnp.int32, 16) * 2 + (ebase + j * 2)

        def issue_g(b):
            pltpu.async_copy(tabs[b].at[gidx.at[b]], buf.at[b], gsem[b])

        def wait_g(b):
            pltpu.make_async_copy(tabs[b].at[gidx.at[b]], buf.at[b], gsem[b]).wait()

        def issue_sc(b):
            pltpu.async_copy(buf.at[b], acc.at[didx.at[b]], ssem[b], add=True)

        def wait_sc(b):
            pltpu.make_async_copy(buf.at[b], acc.at[didx.at[b]], ssem[b]).wait()

        issue_ld(0, 0)
        issue_ld(1, 1)

        @pl.loop(0, _B)
        def _(r):
            @pl.loop(0, h, step=16)
            def _(j):
                buf[0, r, pl.ds(j, 16)] = jnp.zeros((16,), jnp.float32)

        nfull = rows_pt // _B
        ztail = rows_pt - nfull * _B
        for m in range(nfull):
            pltpu.async_copy(buf.at[0], acc.at[pl.ds(arow + m * _B, _B)], zsem)
        if ztail:
            pltpu.async_copy(buf.at[0, pl.ds(0, ztail)],
                             acc.at[pl.ds(arow + nfull * _B, ztail)], zsem)
        for m in range(nfull):
            pltpu.make_async_copy(buf.at[0],
                                  acc.at[pl.ds(arow + m * _B, _B)], zsem).wait()
        if ztail:
            pltpu.make_async_copy(buf.at[0, pl.ds(0, ztail)],
                                  acc.at[pl.ds(arow + nfull * _B, ztail)],
                                  zsem).wait()
        wait_ld(0)
        gen_node(0, 0)
        issue_g(0)
        gen_edge(1, 0, 0)
        issue_g(1)
        issue_ld(2, 0)
        wait_ld(1)
        gen_node(2, 1)
        issue_g(2)
        gen_edge(3, 1, 1)
        issue_g(3)
        issue_ld(3, 1)
        plsc.subcore_barrier()
        wait_g(0)
        issue_sc(0)
        wait_g(1)
        issue_sc(1)
        wait_g(2)
        issue_sc(2)

        @pl.loop(4, njobs - 2, step=4)
        def _(g0_tr):
            kbase = g0_tr // 2
            for b in range(4):
                k_tr = kbase + (b // 2)
                st = b // 2
                if b % 2 == 0:
                    wait_sc(b)
                    wait_ld(st)
                    gen_node(b, st)
                    issue_g(b)
                else:
                    wait_sc(b)
                    gen_edge(b, st, k_tr)
                    issue_g(b)

                    @pl.when(k_tr + 2 < nchunks)
                    def _():
                        issue_ld(k_tr + 2, st)
                bp = (b + 3) % 4
                wait_g(bp)
                issue_sc(bp)

        wait_sc(0)
        wait_ld(0)
        gen_node(0, 0)
        issue_g(0)
        wait_g(3)
        issue_sc(3)
        wait_sc(1)
        gen_edge(1, 0, nchunks - 1)
        issue_g(1)
        wait_g(0)
        issue_sc(0)
        wait_g(1)
        issue_sc(1)
        wait_sc(2)
        wait_sc(3)
        wait_sc(0)
        wait_sc(1)

        plsc.subcore_barrier()
        pltpu.sync_copy(acc.at[pl.ds(arow, rows_pt)],
                        out_hbm.at[c].at[pl.ds(arow, rows_pt)])

    out3 = k(node2, edge2, edge_index.reshape(2 * e))
    return out3[:, :n, :].transpose(1, 0, 2).reshape(n, d)


# speedup vs baseline: 5.0683x; 1.5814x over previous
import functools

import jax
import jax.numpy as jnp
from jax import lax
from jax.experimental import pallas as pl
from jax.experimental.pallas import tpu as pltpu
from jax.experimental.pallas import tpu_sc as plsc

_TILES = 16
_B = 80


def kernel(node_feat, edge_feat, edge_index):
    n, d = node_feat.shape
    e = edge_feat.shape[0]
    h = d // 2

    node2 = node_feat.reshape(2 * n, h)

    epw = e // _TILES
    nchunks = epw // _B
    njobs = 2 * nchunks
    n_pad = ((n + _TILES * 8 - 1) // (_TILES * 8)) * (_TILES * 8)
    rows_pt = n_pad // _TILES
    rows_last = n - (_TILES - 1) * rows_pt

    mesh = plsc.VectorSubcoreMesh(core_axis_name="c", subcore_axis_name="s")

    @functools.partial(
        pl.kernel,
        out_type=jax.ShapeDtypeStruct((2, n_pad, h), jnp.float32),
        mesh=mesh,
        scratch_types=[
            pltpu.VMEM((2, _B), jnp.int32),
            pltpu.VMEM((2, _B), jnp.int32),
            pltpu.VMEM((4, _B), jnp.int32),
            pltpu.VMEM((4, _B), jnp.int32),
            pltpu.VMEM((4, _B, h), jnp.float32),
            pltpu.VMEM_SHARED((n_pad, h), jnp.float32),
            pltpu.SemaphoreType.DMA,
        ]
        + [pltpu.SemaphoreType.DMA] * 4
        + [pltpu.SemaphoreType.DMA] * 4
        + [pltpu.SemaphoreType.DMA] * 2,
    )
    def k(node_hbm, edge_hbm, eidx_hbm, out_hbm,
          sstage, dstage, gidx, didx, buf, acc, zsem,
          g0, g1, g2, g3, s0, s1, s2, s3, l0, l1):
        gsem = [g0, g1, g2, g3]
        ssem = [s0, s1, s2, s3]
        lsem = [l0, l1]

        c = lax.axis_index("c")
        s = lax.axis_index("s")
        base0 = s * epw
        arow = s * rows_pt
        colbase = pl.multiple_of(c * h, h)

        def issue_ld(k_tr, st):
            pltpu.async_copy(eidx_hbm.at[pl.ds(base0 + k_tr * _B, _B)],
                             sstage.at[st], lsem[st])
            pltpu.async_copy(eidx_hbm.at[pl.ds(e + base0 + k_tr * _B, _B)],
                             dstage.at[st], lsem[st])

        def wait_ld(st):
            pltpu.make_async_copy(eidx_hbm.at[pl.ds(base0, _B)],
                                  sstage.at[st], lsem[st]).wait()
            pltpu.make_async_copy(eidx_hbm.at[pl.ds(e + base0, _B)],
                                  dstage.at[st], lsem[st]).wait()

        def gen_node(b, st):
            @pl.loop(0, _B, step=16)
            def _(j):
                didx[b, pl.ds(j, 16)] = dstage[st, pl.ds(j, 16)]
                gidx[b, pl.ds(j, 16)] = sstage[st, pl.ds(j, 16)] * 2 + c

        def gen_edge(b, st, k_tr):
            @pl.loop(0, _B, step=16)
            def _(j):
                didx[b, pl.ds(j, 16)] = dstage[st, pl.ds(j, 16)]

        def issue_g(b, k_tr):
            if b % 2 == 0:
                pltpu.async_copy(node_hbm.at[gidx.at[b]], buf.at[b], gsem[b])
            else:
                pltpu.async_copy(
                    edge_hbm.at[pl.ds(base0 + k_tr * _B, _B), pl.ds(colbase, h)],
                    buf.at[b], gsem[b])

        def wait_g(b):
            if b % 2 == 0:
                pltpu.make_async_copy(node_hbm.at[gidx.at[b]],
                                      buf.at[b], gsem[b]).wait()
            else:
                pltpu.make_async_copy(
                    edge_hbm.at[pl.ds(base0, _B), pl.ds(colbase, h)],
                    buf.at[b], gsem[b]).wait()

        def issue_sc(b):
            pltpu.async_copy(buf.at[b], acc.at[didx.at[b]], ssem[b], add=True)

        def wait_sc(b):
            pltpu.make_async_copy(buf.at[b], acc.at[didx.at[b]], ssem[b]).wait()

        issue_ld(0, 0)
        issue_ld(1, 1)

        @pl.loop(0, _B)
        def _(r):
            @pl.loop(0, h, step=16)
            def _(j):
                buf[0, r, pl.ds(j, 16)] = jnp.zeros((16,), jnp.float32)

        nfull = rows_pt // _B
        ztail = rows_pt - nfull * _B
        for m in range(nfull):
            pltpu.async_copy(buf.at[0], acc.at[pl.ds(arow + m * _B, _B)], zsem)
        if ztail:
            pltpu.async_copy(buf.at[0, pl.ds(0, ztail)],
                             acc.at[pl.ds(arow + nfull * _B, ztail)], zsem)
        for m in range(nfull):
            pltpu.make_async_copy(buf.at[0],
                                  acc.at[pl.ds(arow + m * _B, _B)], zsem).wait()
        if ztail:
            pltpu.make_async_copy(buf.at[0, pl.ds(0, ztail)],
                                  acc.at[pl.ds(arow + nfull * _B, ztail)],
                                  zsem).wait()
        wait_ld(0)
        gen_node(0, 0)
        issue_g(0, 0)
        gen_edge(1, 0, 0)
        issue_g(1, 0)
        issue_ld(2, 0)
        wait_ld(1)
        gen_node(2, 1)
        issue_g(2, 1)
        gen_edge(3, 1, 1)
        issue_g(3, 1)
        issue_ld(3, 1)
        plsc.subcore_barrier()
        wait_g(0)
        issue_sc(0)
        wait_g(1)
        issue_sc(1)
        wait_g(2)
        issue_sc(2)

        @pl.loop(4, njobs - 2, step=4)
        def _(g0_tr):
            kbase = g0_tr // 2
            for b in range(4):
                k_tr = kbase + (b // 2)
                st = b // 2
                if b % 2 == 0:
                    wait_sc(b)
                    wait_ld(st)
                    gen_node(b, st)
                    issue_g(b, k_tr)
                else:
                    wait_sc(b)
                    gen_edge(b, st, k_tr)
                    issue_g(b, k_tr)

                    @pl.when(k_tr + 2 < nchunks)
                    def _():
                        issue_ld(k_tr + 2, st)
                bp = (b + 3) % 4
                wait_g(bp)
                issue_sc(bp)

        wait_sc(0)
        wait_ld(0)
        gen_node(0, 0)
        issue_g(0, nchunks - 1)
        wait_g(3)
        issue_sc(3)
        wait_sc(1)
        gen_edge(1, 0, nchunks - 1)
        issue_g(1, nchunks - 1)
        wait_g(0)
        issue_sc(0)
        wait_g(1)
        issue_sc(1)
        wait_sc(2)
        wait_sc(3)
        wait_sc(0)
        wait_sc(1)

        plsc.subcore_barrier()
        pltpu.sync_copy(acc.at[pl.ds(arow, rows_pt)],
                        out_hbm.at[c].at[pl.ds(arow, rows_pt)])

    out3 = k(node2, edge_feat, edge_index.reshape(2 * e))
    return out3[:, :n, :].transpose(1, 0, 2).reshape(n, d)
